# Initial kernel scaffold; baseline (speedup 1.0000x reference)
import jax
import jax.numpy as jnp
from jax.experimental import pallas as pl


def kernel(x, edge_index, theta1, W1, theta2, W2):
    raise NotImplementedError("write your pallas kernel here")


# speedup vs baseline: 6.3908x; 6.3908x over previous
import functools

import jax
import jax.numpy as jnp
from jax import lax
from jax.experimental import pallas as pl
from jax.experimental.pallas import tpu as pltpu
from jax.experimental.pallas import tpu_sc as plsc

N = 10000
E = 320000
D = 512
HEADS = 8
HIDDEN = 64
K = 16
C = 16
CN = 4
CW = 128
NSC = 2
TPS = 16
RPTA = 624
ROFF = TPS * RPTA
REM = N - ROFF
NB = 10
RB = N // NB

F32 = jnp.float32

EPT = E // TPS
B1 = 128
NFULL1 = EPT // B1
TAIL1 = EPT - NFULL1 * B1


def _apply512_body(srcoff, dstidx, zeros, table, out,
                   acc, idx_g, idx_s, idx_gt, idx_st, rows, rows_t, sem):
    c = lax.axis_index("c")
    s = lax.axis_index("s")
    r0 = pl.multiple_of(s * RPTA, 8)
    last = s == TPS - 1

    def rows_copy(src_at, dst_at):
        pltpu.sync_copy(src_at(r0, RPTA), dst_at(r0, RPTA))

        @pl.when(last)
        def _():
            pltpu.sync_copy(src_at(ROFF, REM), dst_at(ROFF, REM))

    rows_copy(lambda o, n: zeros.at[pl.ds(o, n)],
              lambda o, n: acc.at[pl.ds(o, n)])
    plsc.subcore_barrier()
    for cc in range(2):
        ci = 2 * c + cc
        ebase = ci * E + s * EPT

        def body(j, carry):
            off = pl.multiple_of(ebase + j * B1, 8)
            pltpu.sync_copy(srcoff.at[pl.ds(off, B1)], idx_g)
            off2 = pl.multiple_of(s * EPT + j * B1, 8)
            pltpu.sync_copy(dstidx.at[pl.ds(off2, B1)], idx_s)
            pltpu.async_copy(table.at[idx_g], rows, sem).wait()
            pltpu.sync_copy(rows, acc.at[idx_s], add=True)
            return carry

        lax.fori_loop(0, NFULL1, body, 0)
        offt = pl.multiple_of(ebase + NFULL1 * B1, 8)
        pltpu.sync_copy(srcoff.at[pl.ds(offt, TAIL1)], idx_gt)
        offt2 = pl.multiple_of(s * EPT + NFULL1 * B1, 8)
        pltpu.sync_copy(dstidx.at[pl.ds(offt2, TAIL1)], idx_st)
        pltpu.async_copy(table.at[idx_gt], rows_t, sem).wait()
        pltpu.sync_copy(rows_t, acc.at[idx_st], add=True)
        plsc.subcore_barrier()
        rows_copy(lambda o, n: acc.at[pl.ds(o, n)],
                  lambda o, n: out.at[pl.ds(pl.multiple_of(ci * N + o, 8), n)])
        if cc == 0:
            rows_copy(lambda o, n: zeros.at[pl.ds(o, n)],
                      lambda o, n: acc.at[pl.ds(o, n)])
            plsc.subcore_barrier()


_apply512 = pl.kernel(
    _apply512_body,
    out_type=jax.ShapeDtypeStruct((CN * N, CW), F32),
    mesh=plsc.VectorSubcoreMesh(core_axis_name="c", subcore_axis_name="s"),
    scratch_types=[
        pltpu.VMEM_SHARED((N, CW), F32),
        pltpu.VMEM((B1,), jnp.int32),
        pltpu.VMEM((B1,), jnp.int32),
        pltpu.VMEM((TAIL1,), jnp.int32),
        pltpu.VMEM((TAIL1,), jnp.int32),
        pltpu.VMEM((B1, CW), F32),
        pltpu.VMEM((TAIL1, CW), F32),
        pltpu.SemaphoreType.DMA,
    ],
)

EPS = E // NSC
EPT2 = EPS // TPS
B2 = 128
NFULL2 = EPT2 // B2
TAIL2 = EPT2 - NFULL2 * B2


def _apply16_body(src, dst, zeros16, table, out,
                  acc, idx_g, idx_s, idx_gt, idx_st, rows, rows_t, sem):
    c = lax.axis_index("c")
    s = lax.axis_index("s")
    r0 = pl.multiple_of(s * RPTA, 8)
    last = s == TPS - 1

    def rows_copy(src_at, dst_at):
        pltpu.sync_copy(src_at(r0, RPTA), dst_at(r0, RPTA))

        @pl.when(last)
        def _():
            pltpu.sync_copy(src_at(ROFF, REM), dst_at(ROFF, REM))

    rows_copy(lambda o, n: zeros16.at[pl.ds(o, n)],
              lambda o, n: acc.at[pl.ds(o, n)])
    plsc.subcore_barrier()
    ebase = c * EPS + s * EPT2

    def body(j, carry):
        off = pl.multiple_of(ebase + j * B2, 8)
        pltpu.sync_copy(src.at[pl.ds(off, B2)], idx_g)
        pltpu.sync_copy(dst.at[pl.ds(off, B2)], idx_s)
        pltpu.async_copy(table.at[idx_g], rows, sem).wait()
        pltpu.sync_copy(rows, acc.at[idx_s], add=True)
        return carry

    lax.fori_loop(0, NFULL2, body, 0)
    offt = pl.multiple_of(ebase + NFULL2 * B2, 8)
    pltpu.sync_copy(src.at[pl.ds(offt, TAIL2)], idx_gt)
    pltpu.sync_copy(dst.at[pl.ds(offt, TAIL2)], idx_st)
    pltpu.async_copy(table.at[idx_gt], rows_t, sem).wait()
    pltpu.sync_copy(rows_t, acc.at[idx_st], add=True)
    plsc.subcore_barrier()
    rows_copy(lambda o, n: acc.at[pl.ds(o, n)],
              lambda o, n: out.at[pl.ds(pl.multiple_of(c * N + o, 8), n)])


_apply16 = pl.kernel(
    _apply16_body,
    out_type=jax.ShapeDtypeStruct((NSC * N, C), F32),
    mesh=plsc.VectorSubcoreMesh(core_axis_name="c", subcore_axis_name="s"),
    scratch_types=[
        pltpu.VMEM_SHARED((N, C), F32),
        pltpu.VMEM((B2,), jnp.int32),
        pltpu.VMEM((B2,), jnp.int32),
        pltpu.VMEM((TAIL2,), jnp.int32),
        pltpu.VMEM((TAIL2,), jnp.int32),
        pltpu.VMEM((B2, C), F32),
        pltpu.VMEM((TAIL2, C), F32),
        pltpu.SemaphoreType.DMA,
    ],
    compiler_params=pltpu.CompilerParams(use_tc_tiling_on_sc=False),
)


def _scalars_body(p_ref, a_ref, di_ref, ds_ref):
    p = p_ref[...]
    deg = p[0, :, 0:1] + p[1, :, 0:1]
    dstar = jnp.maximum(deg, 1.0)
    a_ref[...] = 1.0 / dstar
    di_ref[...] = lax.rsqrt(dstar)
    ds_ref[...] = jnp.sqrt(dstar)


_scalars = pl.pallas_call(
    _scalars_body,
    grid=(NB,),
    in_specs=[pl.BlockSpec((NSC, RB, C), lambda i: (0, i, 0))],
    out_specs=[pl.BlockSpec((RB, 1), lambda i: (i, 0))] * 3,
    out_shape=[jax.ShapeDtypeStruct((N, 1), F32)] * 3,
)


def _mm1_body(x_ref, w_ref, di_ref, o_ref):
    y = jnp.dot(x_ref[...], w_ref[...], preferred_element_type=F32)
    y = y * di_ref[...]
    for ci in range(CN):
        o_ref[ci] = y[:, ci * CW:(ci + 1) * CW]


_mm1 = pl.pallas_call(
    _mm1_body,
    grid=(NB,),
    in_specs=[
        pl.BlockSpec((RB, D), lambda i: (i, 0)),
        pl.BlockSpec((D, D), lambda i: (0, 0)),
        pl.BlockSpec((RB, 1), lambda i: (i, 0)),
    ],
    out_specs=pl.BlockSpec((CN, RB, CW), lambda i: (0, i, 0)),
    out_shape=jax.ShapeDtypeStruct((CN, N, CW), F32),
)


def _comb512_first_body(s_ref, tp_ref, a_ref, c0_ref, c1_ref, tn_ref, on_ref):
    tn = -(a_ref[...] * s_ref[...])
    on_ref[...] = c0_ref[0] * tp_ref[...] + c1_ref[0] * tn
    tn_ref[...] = tn


_comb512_first = pl.pallas_call(
    _comb512_first_body,
    grid=(CN * NB,),
    in_specs=[
        pl.BlockSpec((RB, CW), lambda i: (i, 0)),
        pl.BlockSpec((RB, CW), lambda i: (i, 0)),
        pl.BlockSpec((RB, 1), lambda i: (i % NB, 0)),
        pl.BlockSpec((1, 1, CW), lambda i: (i // NB, 0, 0)),
        pl.BlockSpec((1, 1, CW), lambda i: (i // NB, 0, 0)),
    ],
    out_specs=[pl.BlockSpec((RB, CW), lambda i: (i, 0))] * 2,
    out_shape=[jax.ShapeDtypeStruct((CN * N, CW), F32)] * 2,
)


def _comb512_rest_body(s_ref, tp_ref, oin_ref, a_ref, ck_ref, tn_ref, on_ref):
    tn = -2.0 * (a_ref[...] * s_ref[...]) - tp_ref[...]
    on_ref[...] = oin_ref[...] + ck_ref[0] * tn
    tn_ref[...] = tn


_comb512_rest = pl.pallas_call(
    _comb512_rest_body,
    grid=(CN * NB,),
    in_specs=[
        pl.BlockSpec((RB, CW), lambda i: (i, 0)),
        pl.BlockSpec((RB, CW), lambda i: (i, 0)),
        pl.BlockSpec((RB, CW), lambda i: (i, 0)),
        pl.BlockSpec((RB, 1), lambda i: (i % NB, 0)),
        pl.BlockSpec((1, 1, CW), lambda i: (i // NB, 0, 0)),
    ],
    out_specs=[pl.BlockSpec((RB, CW), lambda i: (i, 0))] * 2,
    out_shape=[jax.ShapeDtypeStruct((CN * N, CW), F32)] * 2,
)


def _mm2_body(x_ref, ds_ref, di_ref, w_ref, o_ref):
    dsb = ds_ref[...]
    acc = jnp.zeros((RB, C), F32)
    for ci in range(CN):
        h = dsb * x_ref[ci]
        h = jnp.where(h > 0, h, jnp.exp(h) - 1.0)
        acc = acc + jnp.dot(h, w_ref[ci], preferred_element_type=F32)
    o_ref[...] = di_ref[...] * acc


_mm2 = pl.pallas_call(
    _mm2_body,
    grid=(NB,),
    in_specs=[
        pl.BlockSpec((CN, RB, CW), lambda i: (0, i, 0)),
        pl.BlockSpec((RB, 1), lambda i: (i, 0)),
        pl.BlockSpec((RB, 1), lambda i: (i, 0)),
        pl.BlockSpec((CN, CW, C), lambda i: (0, 0, 0)),
    ],
    out_specs=pl.BlockSpec((RB, C), lambda i: (i, 0)),
    out_shape=jax.ShapeDtypeStruct((N, C), F32),
)


def _comb16_first_body(sp_ref, tp_ref, a_ref, c0_ref, c1_ref, tn_ref, on_ref):
    sv = sp_ref[0] + sp_ref[1]
    tn = -(a_ref[...] * sv)
    on_ref[...] = c0_ref[...] * tp_ref[...] + c1_ref[...] * tn
    tn_ref[...] = tn


_comb16_first = pl.pallas_call(
    _comb16_first_body,
    grid=(NB,),
    in_specs=[
        pl.BlockSpec((NSC, RB, C), lambda i: (0, i, 0)),
        pl.BlockSpec((RB, C), lambda i: (i, 0)),
        pl.BlockSpec((RB, 1), lambda i: (i, 0)),
        pl.BlockSpec((1, C), lambda i: (0, 0)),
        pl.BlockSpec((1, C), lambda i: (0, 0)),
    ],
    out_specs=[pl.BlockSpec((RB, C), lambda i: (i, 0))] * 2,
    out_shape=[jax.ShapeDtypeStruct((N, C), F32)] * 2,
)


def _comb16_rest_body(sp_ref, tp_ref, oin_ref, a_ref, ck_ref, tn_ref, on_ref):
    sv = sp_ref[0] + sp_ref[1]
    tn = -2.0 * (a_ref[...] * sv) - tp_ref[...]
    on_ref[...] = oin_ref[...] + ck_ref[...] * tn
    tn_ref[...] = tn


_comb16_rest = pl.pallas_call(
    _comb16_rest_body,
    grid=(NB,),
    in_specs=[
        pl.BlockSpec((NSC, RB, C), lambda i: (0, i, 0)),
        pl.BlockSpec((RB, C), lambda i: (i, 0)),
        pl.BlockSpec((RB, C), lambda i: (i, 0)),
        pl.BlockSpec((RB, 1), lambda i: (i, 0)),
        pl.BlockSpec((1, C), lambda i: (0, 0)),
    ],
    out_specs=[pl.BlockSpec((RB, C), lambda i: (i, 0))] * 2,
    out_shape=[jax.ShapeDtypeStruct((N, C), F32)] * 2,
)


def _final_body(o2_ref, ds_ref, out_ref, h2_ref):
    h2 = ds_ref[...] * o2_ref[...]
    e = jnp.where(h2 > 0, h2, jnp.exp(h2) - 1.0)
    m = jnp.max(e, axis=1, keepdims=True)
    lse = jnp.log(jnp.sum(jnp.exp(e - m), axis=1, keepdims=True)) + m
    out_ref[...] = e - lse
    h2_ref[...] = h2


_final = pl.pallas_call(
    _final_body,
    grid=(NB,),
    in_specs=[
        pl.BlockSpec((RB, C), lambda i: (i, 0)),
        pl.BlockSpec((RB, 1), lambda i: (i, 0)),
    ],
    out_specs=[pl.BlockSpec((RB, C), lambda i: (i, 0))] * 2,
    out_shape=[jax.ShapeDtypeStruct((N, C), F32)] * 2,
)


def kernel(x, edge_index, theta1, W1, theta2, W2):
    src = edge_index[0]
    dst = edge_index[1]
    srcoff = (src[None, :]
              + (jnp.arange(CN, dtype=jnp.int32) * N)[:, None]).reshape(-1)
    zeros512 = jnp.zeros((N, CW), F32)
    zeros16 = jnp.zeros((N, C), F32)
    ones16 = jnp.ones((N, C), F32)

    degp = _apply16(src, dst, zeros16, ones16)
    a, d_isqrt, d_sqrt = _scalars(degp.reshape(NSC, N, C))

    Wc = W1.transpose(1, 0, 2).reshape(D, D)
    p0f = _mm1(x, Wc, d_isqrt).reshape(CN * N, CW)
    C1 = jnp.repeat(theta1.T, HIDDEN, axis=1).reshape(K, CN, 1, CW)
    s1 = _apply512(srcoff, dst, zeros512, p0f)
    t_cur, out1 = _comb512_first(s1, p0f, a, C1[0], C1[1])
    t_prev = p0f
    for k in range(2, K):
        sk = _apply512(srcoff, dst, zeros512, t_cur)
        t_next, out1 = _comb512_rest(sk, t_prev, out1, a, C1[k])
        t_prev, t_cur = t_cur, t_next

    W2c = W2[0].reshape(CN, CW, C)
    p0b = _mm2(out1.reshape(CN, N, CW), d_sqrt, d_isqrt, W2c)
    C2 = jnp.repeat(theta2.T, C, axis=1)
    s2 = _apply16(src, dst, zeros16, p0b)
    t_cur2, out2 = _comb16_first(s2.reshape(NSC, N, C), p0b, a,
                                 C2[0:1], C2[1:2])
    t_prev2 = p0b
    for k in range(2, K):
        sk2 = _apply16(src, dst, zeros16, t_cur2)
        t_next2, out2 = _comb16_rest(sk2.reshape(NSC, N, C), t_prev2, out2,
                                     a, C2[k:k + 1])
        t_prev2, t_cur2 = t_cur2, t_next2

    out, h2 = _final(out2, d_sqrt)
    return (out, h2)
